# baseline (speedup 1.0000x reference)
import functools

import jax
import jax.numpy as jnp
from jax import lax
from jax.experimental import pallas as pl
from jax.experimental.pallas import tpu as pltpu
from jax.experimental.pallas import tpu_sc as plsc

N_TOK = 32768
DIM = 256
K = 256
BLK = 2048

NC = 2
NS = 16
NW = NC * NS
B_PER_W = N_TOK // NW
CHUNK = 128
NCHUNK = B_PER_W // CHUNK
CH2 = 64
NCH2 = B_PER_W // CH2


def _tc_body(x_ref, w_ref, idx_ref):
    x = x_ref[...]
    w = w_ref[...]
    wt2 = jnp.sum(w * w, axis=0, keepdims=True)
    x2 = jnp.sum(x * x, axis=1, keepdims=True)
    cross = jax.lax.dot_general(
        x, w, (((1,), (0,)), ((), ())),
        preferred_element_type=jnp.float32,
    )
    dist = x2 + wt2 - 2.0 * cross
    m = jnp.min(dist, axis=1, keepdims=True)
    iota = jax.lax.broadcasted_iota(jnp.int32, dist.shape, 1).astype(jnp.float32)
    idx = jnp.min(jnp.where(dist == m, iota, float(K)), axis=1, keepdims=True)
    idx_ref[...] = jnp.reshape(idx.astype(jnp.int32), (BLK // 128, 128))


def _tc_indices(x_flat, W):
    grid = (N_TOK // BLK,)
    return pl.pallas_call(
        _tc_body,
        grid=grid,
        in_specs=[
            pl.BlockSpec((BLK, DIM), lambda i: (i, 0)),
            pl.BlockSpec((DIM, K), lambda i: (0, 0)),
        ],
        out_specs=pl.BlockSpec((BLK // 128, 128), lambda i: (i, 0)),
        out_shape=jax.ShapeDtypeStruct((N_TOK // 128, 128), jnp.int32),
    )(x_flat, W)


_sc_mesh = plsc.VectorSubcoreMesh(core_axis_name="c", subcore_axis_name="s")


@functools.partial(
    pl.kernel,
    out_type=jax.ShapeDtypeStruct((N_TOK * DIM,), jnp.float32),
    mesh=_sc_mesh,
    scratch_types=[
        pltpu.VMEM((NCHUNK, CHUNK), jnp.int32),
        pltpu.VMEM((DIM * K,), jnp.float32),
        pltpu.VMEM((CH2 * DIM,), jnp.float32),
        pltpu.VMEM((CH2 * DIM,), jnp.float32),
        pltpu.SemaphoreType.DMA,
    ],
    compiler_params=pltpu.CompilerParams(needs_layout_passes=False),
)
def _sc_gather(w_hbm, idx_hbm, out_hbm, idx_v, w_v, buf0, buf1, ssem):
    wid = lax.axis_index("s") * NC + lax.axis_index("c")
    base = wid * B_PER_W
    pltpu.sync_copy(w_hbm, w_v)
    pltpu.sync_copy(idx_hbm.at[pl.ds(wid * NCHUNK, NCHUNK)], idx_v)
    bufs = (buf0, buf1)
    lanes = jax.lax.iota(jnp.int32, 16)
    ngrp = CH2 // 16
    dst_bases = [(lanes + g * 16) * DIM for g in range(ngrp)]
    stores = [None] * NCH2

    def build(c, buf):
        row, off = c // 2, (c % 2) * CH2
        src_bases = [
            idx_v[row, pl.ds(off + g * 16, 16)] * DIM for g in range(ngrp)
        ]

        def body(j, _):
            cvec = lanes * 0 + j
            for g in range(ngrp):
                v = plsc.load_gather(w_v, [src_bases[g] + cvec])
                plsc.store_scatter(buf, [dst_bases[g] + cvec], v)
            return 0

        lax.fori_loop(0, DIM, body, 0, unroll=4)

    for c in range(NCH2):
        if c >= 2:
            stores[c - 2].wait()
        build(c, bufs[c % 2])
        stores[c] = pltpu.async_copy(
            bufs[c % 2], out_hbm.at[pl.ds((base + c * CH2) * DIM, CH2 * DIM)],
            ssem)
    stores[NCH2 - 2].wait()
    stores[NCH2 - 1].wait()


@jax.jit
def kernel(x_flat, W):
    idx = _tc_indices(x_flat, W)
    return _sc_gather(W.reshape(-1), idx).reshape(N_TOK, DIM)


# speedup vs baseline: 2.3323x; 2.3323x over previous
import functools

import jax
import jax.numpy as jnp
from jax import lax
from jax.experimental import pallas as pl
from jax.experimental.pallas import tpu as pltpu
from jax.experimental.pallas import tpu_sc as plsc

N_TOK = 32768
DIM = 256
K = 256
BLK = 2048

NC = 2
NS = 16
NW = NC * NS
B_PER_W = N_TOK // NW
CHUNK = 128
NCHUNK = B_PER_W // CHUNK
CH2 = 64
NCH2 = B_PER_W // CH2


def _tc_body(x_ref, w_ref, idx_ref):
    x = x_ref[...]
    w = w_ref[...]
    wt2 = jnp.sum(w * w, axis=0, keepdims=True)
    x2 = jnp.sum(x * x, axis=1, keepdims=True)
    cross = jax.lax.dot_general(
        x, w, (((1,), (0,)), ((), ())),
        preferred_element_type=jnp.float32,
    )
    dist = x2 + wt2 - 2.0 * cross
    m = jnp.min(dist, axis=1, keepdims=True)
    iota = jax.lax.broadcasted_iota(jnp.int32, dist.shape, 1).astype(jnp.float32)
    idx = jnp.min(jnp.where(dist == m, iota, float(K)), axis=1, keepdims=True)
    idx_ref[...] = jnp.reshape(idx.astype(jnp.int32), (BLK // 128, 128))


def _tc_indices(x_flat, W):
    grid = (N_TOK // BLK,)
    return pl.pallas_call(
        _tc_body,
        grid=grid,
        in_specs=[
            pl.BlockSpec((BLK, DIM), lambda i: (i, 0)),
            pl.BlockSpec((DIM, K), lambda i: (0, 0)),
        ],
        out_specs=pl.BlockSpec((BLK // 128, 128), lambda i: (i, 0)),
        out_shape=jax.ShapeDtypeStruct((N_TOK // 128, 128), jnp.int32),
    )(x_flat, W)


_sc_mesh = plsc.VectorSubcoreMesh(core_axis_name="c", subcore_axis_name="s")


WPAD = DIM + 1


@functools.partial(
    pl.kernel,
    out_type=jax.ShapeDtypeStruct((N_TOK * DIM,), jnp.float32),
    mesh=_sc_mesh,
    scratch_types=[
        pltpu.VMEM((B_PER_W,), jnp.int32),
        pltpu.VMEM((K * WPAD,), jnp.float32),
        pltpu.VMEM((CH2 * DIM,), jnp.float32),
        pltpu.VMEM((CH2 * DIM,), jnp.float32),
        pltpu.SemaphoreType.DMA,
    ],
    compiler_params=pltpu.CompilerParams(needs_layout_passes=False),
)
def _sc_gather(w_hbm, idx_hbm, out_hbm, idx_v, w_v, buf0, buf1, ssem):
    wid = lax.axis_index("s") * NC + lax.axis_index("c")
    base = wid * B_PER_W
    pltpu.sync_copy(w_hbm, w_v)
    pltpu.sync_copy(idx_hbm.at[pl.ds(base, B_PER_W)], idx_v)
    bufs = (buf0, buf1)
    lanes = jax.lax.iota(jnp.int32, 16)
    zeros = lanes * 0
    cols = [lanes + j * 16 for j in range(DIM // 16)]
    stores = [None] * NCH2

    def build(c, buf):
        def body(t, _):
            tsplat = plsc.load_gather(idx_v, [zeros + (c * CH2 + t)])
            srcbase = tsplat * WPAD
            for j in range(DIM // 16):
                v = plsc.load_gather(w_v, [srcbase + cols[j]])
                buf[pl.ds(t * DIM + j * 16, 16)] = v
            return 0

        lax.fori_loop(0, CH2, body, 0, unroll=2)

    for c in range(NCH2):
        if c >= 2:
            stores[c - 2].wait()
        build(c, bufs[c % 2])
        stores[c] = pltpu.async_copy(
            bufs[c % 2], out_hbm.at[pl.ds((base + c * CH2) * DIM, CH2 * DIM)],
            ssem)
    stores[NCH2 - 2].wait()
    stores[NCH2 - 1].wait()


@jax.jit
def kernel(x_flat, W):
    idx = _tc_indices(x_flat, W).reshape(-1)
    w_pad = jnp.pad(W, ((0, 0), (0, WPAD - DIM))).reshape(-1)
    return _sc_gather(w_pad, idx).reshape(N_TOK, DIM)


# speedup vs baseline: 3.6902x; 1.5822x over previous
import functools

import jax
import jax.numpy as jnp
from jax import lax
from jax.experimental import pallas as pl
from jax.experimental.pallas import tpu as pltpu
from jax.experimental.pallas import tpu_sc as plsc

N_TOK = 32768
DIM = 256
K = 256
BLK = 2048

NC = 2
NS = 16
NW = NC * NS
B_PER_W = N_TOK // NW
CHUNK = 128
NCHUNK = B_PER_W // CHUNK
CH2 = 64
NCH2 = B_PER_W // CH2


def _tc_body(x_ref, w_ref, idx_ref):
    x = x_ref[...]
    w = w_ref[...]
    wt2 = jnp.sum(w * w, axis=0, keepdims=True)
    x2 = jnp.sum(x * x, axis=1, keepdims=True)
    cross = jax.lax.dot_general(
        x, w, (((1,), (0,)), ((), ())),
        preferred_element_type=jnp.float32,
    )
    dist = x2 + wt2 - 2.0 * cross
    m = jnp.min(dist, axis=1, keepdims=True)
    iota = jax.lax.broadcasted_iota(jnp.int32, dist.shape, 1).astype(jnp.float32)
    idx = jnp.min(jnp.where(dist == m, iota, float(K)), axis=1, keepdims=True)
    idx_ref[...] = jnp.reshape(idx.astype(jnp.int32), (BLK // 128, 128))


def _tc_indices(x_flat, W):
    grid = (N_TOK // BLK,)
    return pl.pallas_call(
        _tc_body,
        grid=grid,
        in_specs=[
            pl.BlockSpec((BLK, DIM), lambda i: (i, 0)),
            pl.BlockSpec((DIM, K), lambda i: (0, 0)),
        ],
        out_specs=pl.BlockSpec((BLK // 128, 128), lambda i: (i, 0)),
        out_shape=jax.ShapeDtypeStruct((N_TOK // 128, 128), jnp.int32),
    )(x_flat, W)


_sc_mesh = plsc.VectorSubcoreMesh(core_axis_name="c", subcore_axis_name="s")


WPAD = DIM + 1


@functools.partial(
    pl.kernel,
    out_type=jax.ShapeDtypeStruct((N_TOK * DIM,), jnp.float32),
    mesh=_sc_mesh,
    scratch_types=[
        pltpu.VMEM((B_PER_W,), jnp.int32),
        pltpu.VMEM((K * WPAD,), jnp.float32),
        pltpu.VMEM((CH2 * DIM,), jnp.float32),
        pltpu.VMEM((CH2 * DIM,), jnp.float32),
        pltpu.SemaphoreType.DMA,
    ],
    compiler_params=pltpu.CompilerParams(needs_layout_passes=False),
)
def _sc_gather(w_hbm, idx_hbm, out_hbm, idx_v, w_v, buf0, buf1, ssem):
    wid = lax.axis_index("s") * NC + lax.axis_index("c")
    base = wid * B_PER_W
    pltpu.sync_copy(w_hbm, w_v)
    pltpu.sync_copy(idx_hbm.at[pl.ds(base, B_PER_W)], idx_v)
    bufs = (buf0, buf1)
    lanes = jax.lax.iota(jnp.int32, 16)
    zeros = lanes * 0
    cols = [lanes + j * 16 for j in range(DIM // 16)]
    stores = [None] * NCH2

    def build(c, buf):
        @plsc.parallel_loop(0, CH2, unroll=4)
        def body(t):
            tsplat = plsc.load_gather(idx_v, [zeros + (c * CH2 + t)])
            srcbase = tsplat * WPAD
            for j in range(DIM // 16):
                v = plsc.load_gather(w_v, [srcbase + cols[j]])
                buf[pl.ds(t * DIM + j * 16, 16)] = v

    for c in range(NCH2):
        if c >= 2:
            stores[c - 2].wait()
        build(c, bufs[c % 2])
        stores[c] = pltpu.async_copy(
            bufs[c % 2], out_hbm.at[pl.ds((base + c * CH2) * DIM, CH2 * DIM)],
            ssem)
    stores[NCH2 - 2].wait()
    stores[NCH2 - 1].wait()


@jax.jit
def kernel(x_flat, W):
    idx = _tc_indices(x_flat, W).reshape(-1)
    w_pad = jnp.pad(W, ((0, 0), (0, WPAD - DIM))).reshape(-1)
    return _sc_gather(w_pad, idx).reshape(N_TOK, DIM)
